# baseline (speedup 1.0000x reference)
import jax
import jax.numpy as jnp
from jax import lax
from jax.experimental import pallas as pl
from jax.experimental.pallas import tpu as pltpu
from jax.experimental.pallas import tpu_sc as plsc

N_USERS = 5000
N_ITEMS = 5000
N_EDGES = 320000
D = 128

NC = 2
NS = 16
L = 16

C = 80
EPT = N_EDGES // NS
NCHUNK = EPT // C
NBUF = 5
QD = NBUF - 2
NOUTER = NCHUNK // NBUF
RC = 200
NRC = N_USERS // RC
ZR = 40
NZC = N_USERS // ZR
PB = 13


def _lane_bcast(wv, e16):
    return lax.gather(
        wv, jnp.full((L, 1), e16, jnp.int32),
        lax.GatherDimensionNumbers(
            offset_dims=(), collapsed_slice_dims=(0,), start_index_map=(0,)),
        slice_sizes=(1,),
        mode=lax.GatherScatterMode.PROMISE_IN_BOUNDS)


def _edge_pass(table_hbm, ew, acc, pbuf, gidx, sidx, wring, rows,
               gsem, ssem, wsem, sid, gather_hi):
    wbase = sid * EPT

    def issue_fetch(q, qb):
        def unpack(g, c2):
            p = pbuf.at[q][pl.ds(g * L, L)]
            hi = lax.shift_right_logical(p, PB)
            lo = lax.bitwise_and(p, (1 << PB) - 1)
            gidx[qb][pl.ds(g * L, L)] = hi if gather_hi else lo
            sidx[qb][pl.ds(g * L, L)] = lo if gather_hi else hi
            return c2

        lax.fori_loop(0, C // L, unpack, 0)
        pltpu.async_copy(table_hbm.at[gidx[qb]], rows[qb], gsem[qb])
        pltpu.async_copy(ew.at[pl.ds(wbase + q * C, C)], wring[qb], wsem[qb])

    def scale(k, b):
        def grp(g, c2):
            wv = wring[b][pl.ds(g * L, L)]
            for e16 in range(L):
                wsp = _lane_bcast(wv, e16)
                e = g * L + e16
                for j in range(D // L):
                    rows[b][e, pl.ds(j * L, L)] = (
                        rows[b][e, pl.ds(j * L, L)] * wsp)
            return c2

        lax.fori_loop(0, C // L, grp, 0, unroll=2)

    def step(k, b):
        pltpu.make_async_copy(table_hbm.at[gidx[b]], rows[b], gsem[b]).wait()
        pltpu.make_async_copy(ew.at[pl.ds(wbase + k * C, C)], wring[b],
                              wsem[b]).wait()
        scale(k, b)
        pltpu.async_copy(rows[b], acc.at[sidx[b]], ssem[b], add=True)
        q = k + QD
        qb = (b + QD) % NBUF

        @pl.when(q < NCHUNK)
        def _():
            @pl.when(k >= NBUF - QD)
            def _():
                pltpu.make_async_copy(rows[qb], acc.at[sidx[qb]],
                                      ssem[qb]).wait()

            issue_fetch(q, qb)

    for b in range(QD):
        issue_fetch(b, b)

    def outer(ko, carry):
        for b in range(NBUF):
            step(ko * NBUF + b, b)
        return carry

    lax.fori_loop(0, NOUTER, outer, 0)

    for b in range(NBUF):
        pltpu.make_async_copy(rows[b], acc.at[sidx[b]], ssem[b]).wait()


def _copy_out(acc, out_hbm, sid):
    for k0 in range(2):
        k = sid + 16 * k0

        @pl.when(k < NRC)
        def _():
            pltpu.sync_copy(acc.at[pl.ds(k * RC, RC)],
                            out_hbm.at[pl.ds(k * RC, RC)])


def _sc_body(h_user, h_item, ew, packed, out_user, out_item, *refs):
    acc, pbuf = refs[0], refs[1]
    rows = list(refs[2:2 + NBUF])
    gidx = list(refs[2 + NBUF:2 + 2 * NBUF])
    sidx = list(refs[2 + 2 * NBUF:2 + 3 * NBUF])
    wring = list(refs[2 + 3 * NBUF:2 + 4 * NBUF])
    gsem = list(refs[2 + 4 * NBUF:2 + 5 * NBUF])
    ssem = list(refs[2 + 5 * NBUF:2 + 6 * NBUF])
    wsem = list(refs[2 + 6 * NBUF:2 + 7 * NBUF])
    cid = lax.axis_index("c")
    sid = lax.axis_index("s")

    pltpu.sync_copy(packed.at[sid], pbuf)

    def zero_body(r, carry):
        for j in range(D // L):
            rows[0][r, pl.ds(j * L, L)] = jnp.zeros((L,), jnp.float32)
        return carry

    lax.fori_loop(0, ZR, zero_body, 0)
    for k0 in range(8):
        k = sid + 16 * k0

        @pl.when(k < NZC)
        def _():
            pltpu.sync_copy(rows[0].at[pl.ds(0, ZR)],
                            acc.at[pl.ds(k * ZR, ZR)])

    plsc.subcore_barrier()

    @pl.when(cid == 0)
    def _():
        _edge_pass(h_user, ew, acc, pbuf, gidx, sidx, wring, rows,
                   gsem, ssem, wsem, sid, gather_hi=True)

    @pl.when(cid != 0)
    def _():
        _edge_pass(h_item, ew, acc, pbuf, gidx, sidx, wring, rows,
                   gsem, ssem, wsem, sid, gather_hi=False)

    plsc.subcore_barrier()

    @pl.when(cid == 0)
    def _():
        _copy_out(acc, out_item, sid)

    @pl.when(cid != 0)
    def _():
        _copy_out(acc, out_user, sid)


@jax.jit
def kernel(h_user, h_item, edge_weight, edge_src, edge_dst):
    mesh = plsc.VectorSubcoreMesh(core_axis_name="c", subcore_axis_name="s",
                                  num_cores=NC, num_subcores=NS)
    f = pl.kernel(
        _sc_body,
        out_type=(
            jax.ShapeDtypeStruct((N_USERS, D), jnp.float32),
            jax.ShapeDtypeStruct((N_ITEMS, D), jnp.float32),
        ),
        mesh=mesh,
        scratch_types=(
            [pltpu.VMEM_SHARED((N_ITEMS, D), jnp.float32)]
            + [pltpu.VMEM((NCHUNK, C), jnp.int32)]
            + [pltpu.VMEM((C, D), jnp.float32)] * NBUF
            + [pltpu.VMEM((C,), jnp.int32)] * (2 * NBUF)
            + [pltpu.VMEM((C,), jnp.float32)] * NBUF
            + [pltpu.SemaphoreType.DMA] * (3 * NBUF)
        ),
    )
    packed = (edge_src << PB) | edge_dst
    return f(h_user, h_item, edge_weight, packed.reshape(NS, NCHUNK, C))


# speedup vs baseline: 1.0409x; 1.0409x over previous
import jax
import jax.numpy as jnp
from jax import lax
from jax.experimental import pallas as pl
from jax.experimental.pallas import tpu as pltpu
from jax.experimental.pallas import tpu_sc as plsc

N_USERS = 5000
N_ITEMS = 5000
N_EDGES = 320000
D = 128

NC = 2
NS = 16
L = 16

C = 80
EPT = N_EDGES // NS
NCHUNK = EPT // C
NBUF = 5
QD = 2
NOUTER = NCHUNK // NBUF
RC = 200
NRC = N_USERS // RC
ZR = 40
NZC = N_USERS // ZR
PB = 13


def _lane_bcast(wv, e16):
    return lax.gather(
        wv, jnp.full((L, 1), e16, jnp.int32),
        lax.GatherDimensionNumbers(
            offset_dims=(), collapsed_slice_dims=(0,), start_index_map=(0,)),
        slice_sizes=(1,),
        mode=lax.GatherScatterMode.PROMISE_IN_BOUNDS)


def _edge_pass(table_hbm, ew, acc, pbuf, gidx, sidx, wring, rows,
               gsem, ssem, wsem, sid, gather_hi):
    wbase = sid * EPT

    def issue_fetch(q, qb):
        def unpack(g, c2):
            p = pbuf.at[q][pl.ds(g * L, L)]
            hi = lax.shift_right_logical(p, PB)
            lo = lax.bitwise_and(p, (1 << PB) - 1)
            gidx[qb][pl.ds(g * L, L)] = hi if gather_hi else lo
            sidx[qb][pl.ds(g * L, L)] = lo if gather_hi else hi
            return c2

        lax.fori_loop(0, C // L, unpack, 0)
        pltpu.async_copy(table_hbm.at[gidx[qb]], rows[qb], gsem[qb])
        pltpu.async_copy(ew.at[pl.ds(wbase + q * C, C)], wring[qb], wsem[qb])

    def scale(k, b):
        def grp(g, c2):
            wv = wring[b][pl.ds(g * L, L)]
            for e16 in range(L):
                wsp = _lane_bcast(wv, e16)
                e = g * L + e16
                for j in range(D // L):
                    rows[b][e, pl.ds(j * L, L)] = (
                        rows[b][e, pl.ds(j * L, L)] * wsp)
            return c2

        lax.fori_loop(0, C // L, grp, 0)

    def step(k, b):
        pltpu.make_async_copy(table_hbm.at[gidx[b]], rows[b], gsem[b]).wait()
        pltpu.make_async_copy(ew.at[pl.ds(wbase + k * C, C)], wring[b],
                              wsem[b]).wait()
        scale(k, b)
        pltpu.async_copy(rows[b], acc.at[sidx[b]], ssem[b], add=True)
        q = k + QD
        qb = (b + QD) % NBUF

        @pl.when(q < NCHUNK)
        def _():
            @pl.when(k >= NBUF - QD)
            def _():
                pltpu.make_async_copy(rows[qb], acc.at[sidx[qb]],
                                      ssem[qb]).wait()

            issue_fetch(q, qb)

    for b in range(QD):
        issue_fetch(b, b)

    def outer(ko, carry):
        for b in range(NBUF):
            step(ko * NBUF + b, b)
        return carry

    lax.fori_loop(0, NOUTER, outer, 0)

    for b in range(NBUF):
        pltpu.make_async_copy(rows[b], acc.at[sidx[b]], ssem[b]).wait()


def _copy_out(acc, out_hbm, sid):
    for k0 in range(2):
        k = sid + 16 * k0

        @pl.when(k < NRC)
        def _():
            pltpu.sync_copy(acc.at[pl.ds(k * RC, RC)],
                            out_hbm.at[pl.ds(k * RC, RC)])


def _sc_body(h_user, h_item, ew, packed, out_user, out_item, *refs):
    acc, pbuf = refs[0], refs[1]
    rows = list(refs[2:2 + NBUF])
    gidx = list(refs[2 + NBUF:2 + 2 * NBUF])
    sidx = list(refs[2 + 2 * NBUF:2 + 3 * NBUF])
    wring = list(refs[2 + 3 * NBUF:2 + 4 * NBUF])
    gsem = list(refs[2 + 4 * NBUF:2 + 5 * NBUF])
    ssem = list(refs[2 + 5 * NBUF:2 + 6 * NBUF])
    wsem = list(refs[2 + 6 * NBUF:2 + 7 * NBUF])
    cid = lax.axis_index("c")
    sid = lax.axis_index("s")

    pltpu.sync_copy(packed.at[sid], pbuf)

    def zero_body(r, carry):
        for j in range(D // L):
            rows[0][r, pl.ds(j * L, L)] = jnp.zeros((L,), jnp.float32)
        return carry

    lax.fori_loop(0, ZR, zero_body, 0)
    for k0 in range(8):
        k = sid + 16 * k0

        @pl.when(k < NZC)
        def _():
            pltpu.sync_copy(rows[0].at[pl.ds(0, ZR)],
                            acc.at[pl.ds(k * ZR, ZR)])

    plsc.subcore_barrier()

    @pl.when(cid == 0)
    def _():
        _edge_pass(h_user, ew, acc, pbuf, gidx, sidx, wring, rows,
                   gsem, ssem, wsem, sid, gather_hi=True)

    @pl.when(cid != 0)
    def _():
        _edge_pass(h_item, ew, acc, pbuf, gidx, sidx, wring, rows,
                   gsem, ssem, wsem, sid, gather_hi=False)

    plsc.subcore_barrier()

    @pl.when(cid == 0)
    def _():
        _copy_out(acc, out_item, sid)

    @pl.when(cid != 0)
    def _():
        _copy_out(acc, out_user, sid)


@jax.jit
def kernel(h_user, h_item, edge_weight, edge_src, edge_dst):
    mesh = plsc.VectorSubcoreMesh(core_axis_name="c", subcore_axis_name="s",
                                  num_cores=NC, num_subcores=NS)
    f = pl.kernel(
        _sc_body,
        out_type=(
            jax.ShapeDtypeStruct((N_USERS, D), jnp.float32),
            jax.ShapeDtypeStruct((N_ITEMS, D), jnp.float32),
        ),
        mesh=mesh,
        scratch_types=(
            [pltpu.VMEM_SHARED((N_ITEMS, D), jnp.float32)]
            + [pltpu.VMEM((NCHUNK, C), jnp.int32)]
            + [pltpu.VMEM((C, D), jnp.float32)] * NBUF
            + [pltpu.VMEM((C,), jnp.int32)] * (2 * NBUF)
            + [pltpu.VMEM((C,), jnp.float32)] * NBUF
            + [pltpu.SemaphoreType.DMA] * (3 * NBUF)
        ),
    )
    packed = (edge_src << PB) | edge_dst
    return f(h_user, h_item, edge_weight, packed.reshape(NS, NCHUNK, C))


# speedup vs baseline: 1.1119x; 1.0681x over previous
import jax
import jax.numpy as jnp
from jax import lax
from jax.experimental import pallas as pl
from jax.experimental.pallas import tpu as pltpu
from jax.experimental.pallas import tpu_sc as plsc

N_USERS = 5000
N_ITEMS = 5000
N_EDGES = 320000
D = 128

NC = 2
NS = 16
L = 16

C = 80
EPT = N_EDGES // NS
NCHUNK = EPT // C
NBUF = 5
QD = NBUF - 2
NOUTER = NCHUNK // NBUF
RC = 200
NRC = N_USERS // RC
ZR = 40
NZC = N_USERS // ZR
PB = 13


def _lane_bcast(wv, e16):
    return lax.gather(
        wv, jnp.full((L, 1), e16, jnp.int32),
        lax.GatherDimensionNumbers(
            offset_dims=(), collapsed_slice_dims=(0,), start_index_map=(0,)),
        slice_sizes=(1,),
        mode=lax.GatherScatterMode.PROMISE_IN_BOUNDS)


def _edge_pass(table_hbm, ew, acc, pbuf, gidx, sidx, wring, rows,
               gsem, ssem, wsem, sid, gather_hi):
    wbase = sid * EPT

    def issue_fetch(q, qb):
        def unpack(g, c2):
            p = pbuf.at[q][pl.ds(g * L, L)]
            hi = lax.shift_right_logical(p, PB)
            lo = lax.bitwise_and(p, (1 << PB) - 1)
            gidx[qb][pl.ds(g * L, L)] = hi if gather_hi else lo
            sidx[qb][pl.ds(g * L, L)] = lo if gather_hi else hi
            return c2

        lax.fori_loop(0, C // L, unpack, 0)
        pltpu.async_copy(table_hbm.at[gidx[qb]], rows[qb], gsem[qb])
        pltpu.async_copy(ew.at[pl.ds(wbase + q * C, C)], wring[qb], wsem[qb])

    def scale(k, b):
        def grp(g, c2):
            wv = wring[b][pl.ds(g * L, L)]
            for e16 in range(L):
                wsp = _lane_bcast(wv, e16)
                e = g * L + e16
                for j in range(D // L):
                    rows[b][e, pl.ds(j * L, L)] = (
                        rows[b][e, pl.ds(j * L, L)] * wsp)
            return c2

        lax.fori_loop(0, C // L, grp, 0)

    def step(k, b):
        pltpu.make_async_copy(table_hbm.at[gidx[b]], rows[b], gsem[b]).wait()
        pltpu.make_async_copy(ew.at[pl.ds(wbase + k * C, C)], wring[b],
                              wsem[b]).wait()
        scale(k, b)
        pltpu.async_copy(rows[b], acc.at[sidx[b]], ssem[b], add=True)
        q = k + QD
        qb = (b + QD) % NBUF

        @pl.when(q < NCHUNK)
        def _():
            @pl.when(k >= NBUF - QD)
            def _():
                pltpu.make_async_copy(rows[qb], acc.at[sidx[qb]],
                                      ssem[qb]).wait()

            issue_fetch(q, qb)

    for b in range(QD):
        issue_fetch(b, b)

    def outer(ko, carry):
        for b in range(NBUF):
            step(ko * NBUF + b, b)
        return carry

    lax.fori_loop(0, NOUTER, outer, 0)

    for b in range(NBUF):
        pltpu.make_async_copy(rows[b], acc.at[sidx[b]], ssem[b]).wait()


def _copy_out(acc, out_hbm, sid):
    for k0 in range(2):
        k = sid + 16 * k0

        @pl.when(k < NRC)
        def _():
            pltpu.sync_copy(acc.at[pl.ds(k * RC, RC)],
                            out_hbm.at[pl.ds(k * RC, RC)])


def _sc_body(h_user, h_item, ew, packed, out_user, out_item, *refs):
    acc, pbuf = refs[0], refs[1]
    rows = list(refs[2:2 + NBUF])
    gidx = list(refs[2 + NBUF:2 + 2 * NBUF])
    sidx = list(refs[2 + 2 * NBUF:2 + 3 * NBUF])
    wring = list(refs[2 + 3 * NBUF:2 + 4 * NBUF])
    gsem = list(refs[2 + 4 * NBUF:2 + 5 * NBUF])
    ssem = list(refs[2 + 5 * NBUF:2 + 6 * NBUF])
    wsem = list(refs[2 + 6 * NBUF:2 + 7 * NBUF])
    cid = lax.axis_index("c")
    sid = lax.axis_index("s")

    pltpu.sync_copy(packed.at[sid], pbuf)

    def zero_body(r, carry):
        for j in range(D // L):
            rows[0][r, pl.ds(j * L, L)] = jnp.zeros((L,), jnp.float32)
        return carry

    lax.fori_loop(0, ZR, zero_body, 0)
    for k0 in range(8):
        k = sid + 16 * k0

        @pl.when(k < NZC)
        def _():
            pltpu.sync_copy(rows[0].at[pl.ds(0, ZR)],
                            acc.at[pl.ds(k * ZR, ZR)])

    plsc.subcore_barrier()

    @pl.when(cid == 0)
    def _():
        _edge_pass(h_user, ew, acc, pbuf, gidx, sidx, wring, rows,
                   gsem, ssem, wsem, sid, gather_hi=True)

    @pl.when(cid != 0)
    def _():
        _edge_pass(h_item, ew, acc, pbuf, gidx, sidx, wring, rows,
                   gsem, ssem, wsem, sid, gather_hi=False)

    plsc.subcore_barrier()

    @pl.when(cid == 0)
    def _():
        _copy_out(acc, out_item, sid)

    @pl.when(cid != 0)
    def _():
        _copy_out(acc, out_user, sid)


@jax.jit
def kernel(h_user, h_item, edge_weight, edge_src, edge_dst):
    mesh = plsc.VectorSubcoreMesh(core_axis_name="c", subcore_axis_name="s",
                                  num_cores=NC, num_subcores=NS)
    f = pl.kernel(
        _sc_body,
        out_type=(
            jax.ShapeDtypeStruct((N_USERS, D), jnp.float32),
            jax.ShapeDtypeStruct((N_ITEMS, D), jnp.float32),
        ),
        mesh=mesh,
        scratch_types=(
            [pltpu.VMEM_SHARED((N_ITEMS, D), jnp.float32)]
            + [pltpu.VMEM((NCHUNK, C), jnp.int32)]
            + [pltpu.VMEM((C, D), jnp.float32)] * NBUF
            + [pltpu.VMEM((C,), jnp.int32)] * (2 * NBUF)
            + [pltpu.VMEM((C,), jnp.float32)] * NBUF
            + [pltpu.SemaphoreType.DMA] * (3 * NBUF)
        ),
    )
    packed = (edge_src << PB) | edge_dst
    return f(h_user, h_item, edge_weight, packed.reshape(NS, NCHUNK, C))
